# baseline (speedup 1.0000x reference)
import functools

import jax
import jax.numpy as jnp
from jax import lax
from jax.experimental import pallas as pl
from jax.experimental.pallas import tpu as pltpu
from jax.experimental.pallas import tpu_sc as plsc

B, T = 4096, 200
D = 2
N = B * T
V = 1000000
VPAD = 1000064
ROWS = 250016

_info = plsc.get_sparse_core_info()
NC, NS, L = _info.num_cores, _info.num_subcores, _info.num_lanes
NW = NC * NS
PER_W = N // NW
C = 1280
NCH = PER_W // C
GR = C // L
UNROLL = 4


def _gather_body(idx_hbm, table_hbm, out_hbm,
                 idx_v, out_v,
                 sx_a, sy_a, col_a, sx_b, sy_b, col_b,
                 rx_a, ry_a, rx_b, ry_b,
                 semx_a, semy_a, semx_b, semy_b):
    wid = lax.axis_index("s") * NC + lax.axis_index("c")
    pltpu.sync_copy(idx_hbm.at[:, wid], idx_v)
    lane = lax.iota(jnp.int32, L)

    def prep_group(m, sx_v, sy_v, col_v, j):
        t = m // 128
        c0 = m - t * 128
        v = idx_v[t // 8, pl.ds((t % 8) * 128 + c0, L)]
        sx = lax.shift_right_logical(v, 3) + lax.shift_left(
            lax.shift_right_logical(v, 7), 4)
        sx_v[pl.ds(j, L)] = sx
        sy_v[pl.ds(j, L)] = sx + 16
        col_v[pl.ds(j, L)] = v & 7

    def sel_group(m, col_v, rx_v, ry_v, j):
        rl = lane + j
        cf = col_v[pl.ds(j, L)]
        v0 = plsc.load_gather(rx_v, [rl, cf])
        v1 = plsc.load_gather(ry_v, [rl, cf])
        tq = m // 128
        cs = m - tq * 128
        out_v[tq, pl.ds(cs, L)] = v0
        out_v[tq, pl.ds(cs + 128, L)] = v1

    def prep(c, sx_v, sy_v, col_v):
        def step(i, carry):
            j0 = i * (L * UNROLL)
            for u in range(UNROLL):
                j = j0 + u * L
                prep_group(c * C + j, sx_v, sy_v, col_v, j)
            return carry
        lax.fori_loop(0, GR // UNROLL, step, 0)

    def select(c, col_v, rx_v, ry_v):
        def step(i, carry):
            j0 = i * (L * UNROLL)
            for u in range(UNROLL):
                j = j0 + u * L
                sel_group(c * C + j, col_v, rx_v, ry_v, j)
            return carry
        lax.fori_loop(0, GR // UNROLL, step, 0)

    def fire(sx_v, sy_v, rx_v, ry_v, semx, semy):
        pltpu.async_copy(table_hbm.at[sx_v], rx_v, semx)
        pltpu.async_copy(table_hbm.at[sy_v], ry_v, semy)

    def drain(sx_v, sy_v, rx_v, ry_v, semx, semy):
        pltpu.make_async_copy(table_hbm.at[sx_v], rx_v, semx).wait()
        pltpu.make_async_copy(table_hbm.at[sy_v], ry_v, semy).wait()

    prep(0, sx_a, sy_a, col_a)
    fire(sx_a, sy_a, rx_a, ry_a, semx_a, semy_a)

    def pipe(k2, carry):
        c0 = 2 * k2
        c1 = c0 + 1
        prep(c1, sx_b, sy_b, col_b)
        drain(sx_a, sy_a, rx_a, ry_a, semx_a, semy_a)
        fire(sx_b, sy_b, rx_b, ry_b, semx_b, semy_b)
        select(c0, col_a, rx_a, ry_a)

        @pl.when(c1 + 1 < NCH)
        def _():
            prep(c1 + 1, sx_a, sy_a, col_a)
            fire(sx_a, sy_a, rx_a, ry_a, semx_a, semy_a)

        drain(sx_b, sy_b, rx_b, ry_b, semx_b, semy_b)
        select(c1, col_b, rx_b, ry_b)
        return carry

    lax.fori_loop(0, NCH // 2, pipe, 0)
    pltpu.sync_copy(out_v, out_hbm.at[:, wid])


@jax.jit
def _gather(ids_native, table8):
    mesh = plsc.VectorSubcoreMesh(core_axis_name="c", subcore_axis_name="s")
    run = pl.kernel(
        _gather_body,
        out_type=jax.ShapeDtypeStruct((T, NW, 2 * 128), jnp.float32),
        mesh=mesh,
        scratch_types=[
            pltpu.VMEM((25, 1024), jnp.int32),
            pltpu.VMEM((T, 2 * 128), jnp.float32),
            pltpu.VMEM((C,), jnp.int32),
            pltpu.VMEM((C,), jnp.int32),
            pltpu.VMEM((C,), jnp.int32),
            pltpu.VMEM((C,), jnp.int32),
            pltpu.VMEM((C,), jnp.int32),
            pltpu.VMEM((C,), jnp.int32),
            pltpu.VMEM((C, 8), jnp.float32),
            pltpu.VMEM((C, 8), jnp.float32),
            pltpu.VMEM((C, 8), jnp.float32),
            pltpu.VMEM((C, 8), jnp.float32),
            pltpu.SemaphoreType.DMA,
            pltpu.SemaphoreType.DMA,
            pltpu.SemaphoreType.DMA,
            pltpu.SemaphoreType.DMA,
        ],
        compiler_params=pltpu.CompilerParams(
            use_tc_tiling_on_sc=False, needs_layout_passes=False
        ),
    )
    return run(ids_native, table8)


def kernel(sensor_ids, positions):
    ids = sensor_ids.astype(jnp.int32)
    ids_native = (
        ids.reshape(32, 128, 25, 8)
        .transpose(2, 0, 3, 1)
        .reshape(25, 32, 1024)
    )
    table8 = (
        jnp.pad(positions, ((0, VPAD - V), (0, 0)))
        .reshape(VPAD // 128, 128, 2)
        .transpose(0, 2, 1)
        .reshape(ROWS, 8)
    )
    out3 = _gather(ids_native, table8)
    return (
        out3.reshape(T, NW, 2, 128)
        .transpose(1, 3, 0, 2)
        .reshape(B, T, D)
    )


# speedup vs baseline: 1.2231x; 1.2231x over previous
import functools

import jax
import jax.numpy as jnp
from jax import lax
from jax.experimental import pallas as pl
from jax.experimental.pallas import tpu as pltpu
from jax.experimental.pallas import tpu_sc as plsc

B, T = 4096, 200
D = 2
N = B * T
V = 1000000
VPAD = 1000448
ROWS = 250112
PLANE_ROWS = 125056

_info = plsc.get_sparse_core_info()
NC, NS, L = _info.num_cores, _info.num_subcores, _info.num_lanes
NW = NC * NS
PER_W = N // NW
C = 1280
NCH = PER_W // C
GR = C // L
UNROLL = 4


def _gather_body(idx_hbm, table_hbm, out_hbm,
                 idx_v, out_v,
                 sx_a, sy_a, col_a, sx_b, sy_b, col_b,
                 rx_a, ry_a, rx_b, ry_b,
                 semx_a, semy_a, semx_b, semy_b):
    wid = lax.axis_index("s") * NC + lax.axis_index("c")
    pltpu.sync_copy(idx_hbm.at[:, wid], idx_v)
    lane = lax.iota(jnp.int32, L)

    def prep_group(m, sx_v, sy_v, col_v, j):
        t = m // 128
        c0 = m - t * 128
        v = idx_v[t // 8, pl.ds((t % 8) * 128 + c0, L)]
        sx = lax.shift_right_logical(v, 3)
        sx_v[pl.ds(j, L)] = sx
        sy_v[pl.ds(j, L)] = sx + PLANE_ROWS
        col_v[pl.ds(j, L)] = v & 7

    def sel_group(m, col_v, rx_v, ry_v, j):
        rl = lane + j
        cf = col_v[pl.ds(j, L)]
        v0 = plsc.load_gather(rx_v, [rl, cf])
        v1 = plsc.load_gather(ry_v, [rl, cf])
        tq = m // 128
        cs = m - tq * 128
        out_v[tq, pl.ds(cs, L)] = v0
        out_v[tq, pl.ds(cs + 128, L)] = v1

    def prep(c, sx_v, sy_v, col_v):
        def step(i, carry):
            j0 = i * (L * UNROLL)
            for u in range(UNROLL):
                j = j0 + u * L
                prep_group(c * C + j, sx_v, sy_v, col_v, j)
            return carry
        lax.fori_loop(0, GR // UNROLL, step, 0)

    def select(c, col_v, rx_v, ry_v):
        def step(i, carry):
            j0 = i * (L * UNROLL)
            for u in range(UNROLL):
                j = j0 + u * L
                sel_group(c * C + j, col_v, rx_v, ry_v, j)
            return carry
        lax.fori_loop(0, GR // UNROLL, step, 0)

    def fire(sx_v, sy_v, rx_v, ry_v, semx, semy):
        pltpu.async_copy(table_hbm.at[sx_v], rx_v, semx)
        pltpu.async_copy(table_hbm.at[sy_v], ry_v, semy)

    def drain(sx_v, sy_v, rx_v, ry_v, semx, semy):
        pltpu.make_async_copy(table_hbm.at[sx_v], rx_v, semx).wait()
        pltpu.make_async_copy(table_hbm.at[sy_v], ry_v, semy).wait()

    prep(0, sx_a, sy_a, col_a)
    fire(sx_a, sy_a, rx_a, ry_a, semx_a, semy_a)

    def pipe(k2, carry):
        c0 = 2 * k2
        c1 = c0 + 1
        prep(c1, sx_b, sy_b, col_b)
        drain(sx_a, sy_a, rx_a, ry_a, semx_a, semy_a)
        fire(sx_b, sy_b, rx_b, ry_b, semx_b, semy_b)
        select(c0, col_a, rx_a, ry_a)

        @pl.when(c1 + 1 < NCH)
        def _():
            prep(c1 + 1, sx_a, sy_a, col_a)
            fire(sx_a, sy_a, rx_a, ry_a, semx_a, semy_a)

        drain(sx_b, sy_b, rx_b, ry_b, semx_b, semy_b)
        select(c1, col_b, rx_b, ry_b)
        return carry

    lax.fori_loop(0, NCH // 2, pipe, 0)
    pltpu.sync_copy(out_v, out_hbm.at[:, wid])


@jax.jit
def _gather(ids_native, table8):
    mesh = plsc.VectorSubcoreMesh(core_axis_name="c", subcore_axis_name="s")
    run = pl.kernel(
        _gather_body,
        out_type=jax.ShapeDtypeStruct((T, NW, 2 * 128), jnp.float32),
        mesh=mesh,
        scratch_types=[
            pltpu.VMEM((25, 1024), jnp.int32),
            pltpu.VMEM((T, 2 * 128), jnp.float32),
            pltpu.VMEM((C,), jnp.int32),
            pltpu.VMEM((C,), jnp.int32),
            pltpu.VMEM((C,), jnp.int32),
            pltpu.VMEM((C,), jnp.int32),
            pltpu.VMEM((C,), jnp.int32),
            pltpu.VMEM((C,), jnp.int32),
            pltpu.VMEM((C, 8), jnp.float32),
            pltpu.VMEM((C, 8), jnp.float32),
            pltpu.VMEM((C, 8), jnp.float32),
            pltpu.VMEM((C, 8), jnp.float32),
            pltpu.SemaphoreType.DMA,
            pltpu.SemaphoreType.DMA,
            pltpu.SemaphoreType.DMA,
            pltpu.SemaphoreType.DMA,
        ],
        compiler_params=pltpu.CompilerParams(
            use_tc_tiling_on_sc=False, needs_layout_passes=False
        ),
    )
    return run(ids_native, table8)


def kernel(sensor_ids, positions):
    ids = sensor_ids.astype(jnp.int32)
    ids_native = (
        ids.reshape(32, 128, 25, 8)
        .transpose(2, 0, 3, 1)
        .reshape(25, 32, 1024)
    )
    table8 = (
        jnp.pad(positions, ((0, VPAD - V), (0, 0)))
        .reshape(VPAD // 128, 128, 2)
        .transpose(2, 0, 1)
        .reshape(ROWS, 8)
    )
    out3 = _gather(ids_native, table8)
    return (
        out3.reshape(T, NW, 2, 128)
        .transpose(1, 3, 0, 2)
        .reshape(B, T, D)
    )
